# Initial kernel scaffold; baseline (speedup 1.0000x reference)
import jax
import jax.numpy as jnp
from jax.experimental import pallas as pl


def kernel(x, edge_attr, params, edge_index, invalid_action_mask, action):
    raise NotImplementedError("write your pallas kernel here")


# speedup vs baseline: 1.0704x; 1.0704x over previous
import functools

import jax
import jax.numpy as jnp
from jax.experimental import pallas as pl
from jax.experimental.pallas import tpu as pltpu

_N = 10000
_E = 320000


def _mm_body(x_ref, w_ref, o_ref):
    o_ref[...] = jnp.dot(x_ref[...], w_ref[...],
                         preferred_element_type=jnp.float32)


def _matmul(x, w, bn):
    n, k = x.shape
    m = w.shape[1]
    return pl.pallas_call(
        _mm_body,
        grid=(n // bn,),
        in_specs=[pl.BlockSpec((bn, k), lambda i: (i, 0)),
                  pl.BlockSpec((k, m), lambda i: (0, 0))],
        out_specs=pl.BlockSpec((bn, m), lambda i: (i, 0)),
        out_shape=jax.ShapeDtypeStruct((n, m), jnp.float32),
    )(x, w)


def _layer(p, h_in, src, dst, edge_attr, ha):
    h = _matmul(h_in, p['W'], 1000)
    hs = h @ p['a_src']
    hd = h @ p['a_dst']
    e = hs[src] + hd[dst] + ha
    e = jnp.where(e > 0, e, 0.2 * e)
    ex = jnp.exp(e)
    s = jax.ops.segment_sum(ex, dst, num_segments=_N)
    acc_h = jax.ops.segment_sum(ex[:, None] * h[src], dst, num_segments=_N)
    acc_e = jax.ops.segment_sum(ex[:, None] * edge_attr, dst, num_segments=_N)
    return (acc_h + acc_e @ p['We']) / (s[:, None] + 1e-16)


def _net(layers, x, src, dst, edge_attr, ha_all, base):
    h = x
    for i, p in enumerate(layers):
        h = _layer(p, h, src, dst, edge_attr, ha_all[:, base + i])
        if i < len(layers) - 1:
            h = jax.nn.elu(h)
    return h.mean(axis=0)


def kernel(x, edge_attr, params, edge_index, invalid_action_mask, action):
    src, dst = edge_index[0], edge_index[1]
    w_ha = jnp.stack(
        [p['We'] @ p['a_e'] for p in params['actor']]
        + [p['We'] @ p['a_e'] for p in params['critic']]
        + [jnp.zeros((16,), jnp.float32)] * 2, axis=1)
    ha_all = _matmul(edge_attr, w_ha, 4000)

    crit = [dict(p) for p in params['critic']]
    last = crit[-1]
    pad = lambda a, w: jnp.pad(a, [(0, 0)] * (a.ndim - 1) + [(0, w - a.shape[-1])])
    crit[-1] = {'W': pad(last['W'], 8), 'We': pad(last['We'], 8),
                'a_src': pad(last['a_src'], 8), 'a_dst': pad(last['a_dst'], 8),
                'a_e': last['a_e']}

    logits = _net(params['actor'], x, src, dst, edge_attr, ha_all, 0)
    value = _net(crit, x, src, dst, edge_attr, ha_all, 3)[:1]

    masked = jnp.where(invalid_action_mask, logits, jnp.float32(-1e8))
    logp = jax.nn.log_softmax(masked)
    probs = jnp.exp(logp)
    log_prob = logp[action]
    p_log_p = jnp.where(invalid_action_mask, logp * probs, 0.0)
    entropy = -p_log_p.sum(-1)
    return (jnp.asarray(action), log_prob, entropy, value)


# speedup vs baseline: 12.4158x; 11.5996x over previous
import functools

import jax
import jax.numpy as jnp
from jax import lax
from jax.experimental import pallas as pl
from jax.experimental.pallas import tpu as pltpu
from jax.experimental.pallas import tpu_sc as plsc

_N = 10000
_E = 320000
_C = 80
_NCHUNK = 125
_EP = _C * _NCHUNK
_NTILES = 32


def _mm_body(x_ref, w_ref, o_ref):
    o_ref[...] = jnp.dot(x_ref[...], w_ref[...],
                         preferred_element_type=jnp.float32)


def _matmul(x, w, bn):
    n, k = x.shape
    m = w.shape[1]
    return pl.pallas_call(
        _mm_body,
        grid=(n // bn,),
        in_specs=[pl.BlockSpec((bn, k), lambda i: (i, 0)),
                  pl.BlockSpec((k, m), lambda i: (0, 0))],
        out_specs=pl.BlockSpec((bn, m), lambda i: (i, 0)),
        out_shape=jax.ShapeDtypeStruct((n, m), jnp.float32),
    )(x, w)


def _mm2_body(x_ref, w_ref, wsd_ref, h_ref, sd_ref):
    x = x_ref[...]
    h_ref[...] = jnp.dot(x, w_ref[...], preferred_element_type=jnp.float32)
    sd_ref[...] = jnp.dot(x, wsd_ref[...], preferred_element_type=jnp.float32)


def _mm2(x, w, wsd, bn=1000):
    n, k = x.shape
    m = w.shape[1]
    return pl.pallas_call(
        _mm2_body,
        grid=(n // bn,),
        in_specs=[pl.BlockSpec((bn, k), lambda i: (i, 0)),
                  pl.BlockSpec((k, m), lambda i: (0, 0)),
                  pl.BlockSpec((k, 8), lambda i: (0, 0))],
        out_specs=[pl.BlockSpec((bn, m), lambda i: (i, 0)),
                   pl.BlockSpec((bn, 8), lambda i: (i, 0))],
        out_shape=[jax.ShapeDtypeStruct((n, m), jnp.float32),
                   jax.ShapeDtypeStruct((n, 8), jnp.float32)],
    )(x, w, wsd)


def _combine_body(acch_ref, acce_ref, s_ref, we_ref, wn_ref, wsd_ref,
                  h_ref, sd_ref):
    acc = acch_ref[0] + acch_ref[1]
    acc_e = acce_ref[0] + acce_ref[1]
    s = jnp.sum(s_ref[...], axis=1) + 1e-16
    out = (acc + jnp.dot(acc_e, we_ref[...],
                         preferred_element_type=jnp.float32)) / s[:, None]
    out = jnp.where(out > 0, out, jnp.exp(jnp.minimum(out, 0.0)) - 1.0)
    h_ref[...] = jnp.dot(out, wn_ref[...], preferred_element_type=jnp.float32)
    sd_ref[...] = jnp.dot(out, wsd_ref[...], preferred_element_type=jnp.float32)


def _combine_next(acch, acce, s, we, wn, wsd, bn=1000):
    dh = acch.shape[2]
    m = wn.shape[1]
    return pl.pallas_call(
        _combine_body,
        grid=(_N // bn,),
        in_specs=[pl.BlockSpec((2, bn, dh), lambda i: (0, i, 0)),
                  pl.BlockSpec((2, bn, 16), lambda i: (0, i, 0)),
                  pl.BlockSpec((bn, _NTILES), lambda i: (i, 0)),
                  pl.BlockSpec((16, dh), lambda i: (0, 0)),
                  pl.BlockSpec((dh, m), lambda i: (0, 0)),
                  pl.BlockSpec((dh, 8), lambda i: (0, 0))],
        out_specs=[pl.BlockSpec((bn, m), lambda i: (i, 0)),
                   pl.BlockSpec((bn, 8), lambda i: (i, 0))],
        out_shape=[jax.ShapeDtypeStruct((_N, m), jnp.float32),
                   jax.ShapeDtypeStruct((_N, 8), jnp.float32)],
    )(acch, acce, s, we, wn, wsd)


def _final_body(acch_a, acce_a, s_a, we_a, acch_c, acce_c, s_c, we_c,
                mask_ref, act_ref, lp_ref, ent_ref, val_ref, lacc, vacc):
    i = pl.program_id(0)
    ng = pl.num_programs(0)

    @pl.when(i == 0)
    def _():
        lacc[...] = jnp.zeros_like(lacc)
        vacc[...] = jnp.zeros_like(vacc)

    acc = acch_a[0] + acch_a[1]
    acc_e = acce_a[0] + acce_a[1]
    s = jnp.sum(s_a[...], axis=1) + 1e-16
    out_a = (acc + jnp.dot(acc_e, we_a[...],
                           preferred_element_type=jnp.float32)) / s[:, None]
    lacc[...] += jnp.sum(out_a, axis=0, keepdims=True)

    accc = acch_c[0] + acch_c[1]
    acc_ec = acce_c[0] + acce_c[1]
    sc = jnp.sum(s_c[...], axis=1) + 1e-16
    out_c = (accc + jnp.dot(acc_ec, we_c[...],
                            preferred_element_type=jnp.float32)) / sc[:, None]
    vacc[...] += jnp.sum(out_c, axis=0, keepdims=True)

    @pl.when(i == ng - 1)
    def _():
        logits = lacc[...] / _N
        mask = mask_ref[...] > 0
        masked = jnp.where(mask, logits, jnp.float32(-1e8))
        m = jnp.max(masked)
        lse = jnp.log(jnp.sum(jnp.exp(masked - m))) + m
        logp = masked - lse
        probs = jnp.exp(logp)
        sel = lax.broadcasted_iota(jnp.int32, logp.shape, 1) == act_ref[0]
        lp_ref[...] = jnp.full_like(lp_ref, jnp.sum(jnp.where(sel, logp, 0.0)))
        plp = jnp.where(mask, logp * probs, 0.0)
        ent_ref[...] = jnp.full_like(ent_ref, -jnp.sum(plp))
        val_ref[...] = jnp.full_like(val_ref, vacc[0, 0] / _N)


def _final_head(acch_a, acce_a, s_a, we_a, acch_c, acce_c, s_c, we_c,
                mask_f, action, bn=1000):
    z = jax.ShapeDtypeStruct((1, 1), jnp.float32)
    return pl.pallas_call(
        _final_body,
        grid=(_N // bn,),
        in_specs=[pl.BlockSpec((2, bn, 64), lambda i: (0, i, 0)),
                  pl.BlockSpec((2, bn, 16), lambda i: (0, i, 0)),
                  pl.BlockSpec((bn, _NTILES), lambda i: (i, 0)),
                  pl.BlockSpec((16, 64), lambda i: (0, 0)),
                  pl.BlockSpec((2, bn, 16), lambda i: (0, i, 0)),
                  pl.BlockSpec((2, bn, 16), lambda i: (0, i, 0)),
                  pl.BlockSpec((bn, _NTILES), lambda i: (i, 0)),
                  pl.BlockSpec((16, 16), lambda i: (0, 0)),
                  pl.BlockSpec((1, 64), lambda i: (0, 0)),
                  pl.BlockSpec(memory_space=pltpu.SMEM)],
        out_specs=[pl.BlockSpec((1, 1), lambda i: (0, 0)),
                   pl.BlockSpec((1, 1), lambda i: (0, 0)),
                   pl.BlockSpec((1, 1), lambda i: (0, 0))],
        out_shape=[z, z, z],
        scratch_shapes=[pltpu.VMEM((1, 64), jnp.float32),
                        pltpu.VMEM((1, 16), jnp.float32)],
    )(acch_a, acce_a, s_a, we_a, acch_c, acce_c, s_c, we_c, mask_f, action)


@functools.cache
def _make_sc_layer(dh):
    mesh = plsc.VectorSubcoreMesh(core_axis_name="c", subcore_axis_name="s")

    @functools.partial(
        pl.kernel,
        out_type=(jax.ShapeDtypeStruct((2, _N, dh), jnp.float32),
                  jax.ShapeDtypeStruct((2, _N, 16), jnp.float32),
                  jax.ShapeDtypeStruct((_NTILES, 1, _N), jnp.float32)),
        mesh=mesh,
        compiler_params=pltpu.CompilerParams(needs_layout_passes=False,
                                             use_tc_tiling_on_sc=False),
        scratch_types=(
            pltpu.VMEM((_N,), jnp.float32),
            pltpu.VMEM((_N,), jnp.float32),
            pltpu.VMEM((1, _N), jnp.float32),
            pltpu.VMEM((_NCHUNK, _C), jnp.int32),
            pltpu.VMEM((_NCHUNK, _C), jnp.int32),
            pltpu.VMEM((_NCHUNK, _C), jnp.float32),
            pltpu.VMEM((_NCHUNK, _C), jnp.float32),
            pltpu.VMEM((_C, 16), jnp.float32),
            pltpu.VMEM((_C, dh), jnp.float32),
            pltpu.VMEM_SHARED((_N, dh), jnp.float32),
            pltpu.VMEM_SHARED((_N, 16), jnp.float32),
            pltpu.SemaphoreType.DMA,
        ),
    )
    def sc_layer(h_hbm, hs_hbm, hd_hbm, src_hbm, dst_hbm, ha_hbm, ea_hbm,
                 zh_hbm, ze_hbm, z1_hbm,
                 acch_out, acce_out, s_out,
                 hs_v, hd_v, sl_v, src_v, dst_v, ha_v, ex_v, ea_v, rows_v,
                 acch_sh, acce_sh, sem):
        cid = lax.axis_index("c")
        sid = lax.axis_index("s")
        wid = sid * 2 + cid

        @pl.when(sid == 0)
        def _():
            pltpu.sync_copy(zh_hbm, acch_sh)
            pltpu.sync_copy(ze_hbm, acce_sh)

        pltpu.sync_copy(hs_hbm, hs_v)
        pltpu.sync_copy(hd_hbm, hd_v)
        pltpu.sync_copy(z1_hbm, sl_v)
        pltpu.sync_copy(src_hbm.at[wid], src_v)
        pltpu.sync_copy(dst_hbm.at[wid], dst_v)
        pltpu.sync_copy(ha_hbm.at[wid], ha_v)
        plsc.subcore_barrier()

        def srow(r, _):
            for j in range(_C // 16):
                sl = pl.ds(j * 16, 16)
                vsrc = src_v[r, sl]
                vdst = dst_v[r, sl]
                e = (plsc.load_gather(hs_v, [vsrc])
                     + plsc.load_gather(hd_v, [vdst])
                     + ha_v[r, sl])
                e = jnp.where(e > 0, e, 0.2 * e)
                ex = jnp.exp(e)
                ex_v[r, sl] = ex
                plsc.addupdate_scatter(sl_v, [jnp.zeros((16,), jnp.int32), vdst], ex)
            return 0
        lax.fori_loop(0, _NCHUNK, srow, 0, unroll=False)

        def chunk(k, _):
            pltpu.sync_copy(ea_hbm.at[pl.ds(wid * _EP + k * _C, _C)], ea_v)
            pltpu.async_copy(h_hbm.at[src_v.at[k]], rows_v, sem).wait()

            def scale(r, _):
                a = plsc.load_gather(
                    ex_v, [jnp.zeros((16,), jnp.int32) + k,
                           jnp.zeros((16,), jnp.int32) + r])
                for g in range(dh // 16):
                    gs = pl.ds(g * 16, 16)
                    rows_v[r, gs] = rows_v[r, gs] * a
                ea_v[r, pl.ds(0, 16)] = ea_v[r, pl.ds(0, 16)] * a
                return 0
            lax.fori_loop(0, _C, scale, 0, unroll=False)

            pltpu.sync_copy(rows_v, acch_sh.at[dst_v.at[k]], add=True)
            pltpu.sync_copy(ea_v, acce_sh.at[dst_v.at[k]], add=True)
            return 0
        lax.fori_loop(0, _NCHUNK, chunk, 0, unroll=False)

        pltpu.sync_copy(sl_v, s_out.at[wid])
        plsc.subcore_barrier()

        @pl.when(sid == 0)
        def _():
            pltpu.sync_copy(acch_sh, acch_out.at[cid])
            pltpu.sync_copy(acce_sh, acce_out.at[cid])

    return sc_layer


def _prep_layer(p, width=None):
    w, we = p['W'], p['We']
    asrc, adst, ae = p['a_src'], p['a_dst'], p['a_e']
    if width is not None:
        padw = width - w.shape[1]
        w = jnp.pad(w, ((0, 0), (0, padw)))
        we = jnp.pad(we, ((0, 0), (0, padw)))
        asrc = jnp.pad(asrc, (0, padw))
        adst = jnp.pad(adst, (0, padw))
    wsd = jnp.concatenate(
        [(w @ asrc)[:, None], (w @ adst)[:, None],
         jnp.zeros((w.shape[0], 6), jnp.float32)], axis=1)
    return {'W': w, 'We': we, 'wsd': wsd, 'wha': p['We'] @ p['a_e']}


def kernel(x, edge_attr, params, edge_index, invalid_action_mask, action):
    src = edge_index[0].reshape(_NTILES, _NCHUNK, _C)
    dst = edge_index[1].reshape(_NTILES, _NCHUNK, _C)

    actor = [_prep_layer(p) for p in params['actor']]
    critic = [_prep_layer(p) for p in params['critic'][:2]]
    critic.append(_prep_layer(params['critic'][2], width=16))

    w_ha = jnp.stack([l['wha'] for l in actor] + [l['wha'] for l in critic]
                     + [jnp.zeros((16,), jnp.float32)] * 2, axis=1)
    ha_all = _matmul(edge_attr, w_ha, 4000)

    zeros = {dh: jnp.zeros((_N, dh), jnp.float32) for dh in (16, 64)}
    z1 = jnp.zeros((1, _N), jnp.float32)

    def run_net(layers, base):
        h, sd = _mm2(x, layers[0]['W'], layers[0]['wsd'])
        accs = None
        for i, l in enumerate(layers):
            dh = l['W'].shape[1]
            if i > 0:
                h, sd = _combine_next(*accs, layers[i - 1]['We'],
                                      l['W'], l['wsd'])
            ha = ha_all[:, base + i].reshape(_NTILES, _NCHUNK, _C)
            acch, acce, sp = _make_sc_layer(dh)(
                h, jnp.copy(sd[:, 0]), jnp.copy(sd[:, 1]),
                src, dst, ha, edge_attr, zeros[dh], zeros[16], z1)
            accs = (acch, acce, sp.reshape(_NTILES, _N).T)
        return accs

    acch_a, acce_a, s_a = run_net(actor, 0)
    acch_c, acce_c, s_c = run_net(critic, 3)

    mask_f = invalid_action_mask.astype(jnp.float32).reshape(1, 64)
    act = jnp.asarray(action, jnp.int32).reshape(1)
    lp, ent, val = _final_head(acch_a, acce_a, s_a, actor[2]['We'],
                               acch_c, acce_c, s_c, critic[2]['We'],
                               mask_f, act)
    return (jnp.asarray(action), lp[0, 0], ent[0, 0], val[0])


# speedup vs baseline: 18.7904x; 1.5134x over previous
import functools

import jax
import jax.numpy as jnp
from jax import lax
from jax.experimental import pallas as pl
from jax.experimental.pallas import tpu as pltpu
from jax.experimental.pallas import tpu_sc as plsc

_N = 10000
_E = 320000
_C = 80
_NCHUNK = 125
_EP = _C * _NCHUNK
_NTILES = 32


def _mm_body(x_ref, w_ref, o_ref):
    o_ref[...] = jnp.dot(x_ref[...], w_ref[...],
                         preferred_element_type=jnp.float32)


def _matmul(x, w, bn):
    n, k = x.shape
    m = w.shape[1]
    return pl.pallas_call(
        _mm_body,
        grid=(n // bn,),
        in_specs=[pl.BlockSpec((bn, k), lambda i: (i, 0)),
                  pl.BlockSpec((k, m), lambda i: (0, 0))],
        out_specs=pl.BlockSpec((bn, m), lambda i: (i, 0)),
        out_shape=jax.ShapeDtypeStruct((n, m), jnp.float32),
    )(x, w)


def _mm2_body(x_ref, w_ref, wsd_ref, h_ref, sd_ref):
    x = x_ref[...]
    h_ref[...] = jnp.dot(x, w_ref[...], preferred_element_type=jnp.float32)
    sd_ref[...] = jnp.dot(x, wsd_ref[...], preferred_element_type=jnp.float32)


def _mm2(x, w, wsd, bn=1000):
    n, k = x.shape
    m = w.shape[1]
    return pl.pallas_call(
        _mm2_body,
        grid=(n // bn,),
        in_specs=[pl.BlockSpec((bn, k), lambda i: (i, 0)),
                  pl.BlockSpec((k, m), lambda i: (0, 0)),
                  pl.BlockSpec((k, 8), lambda i: (0, 0))],
        out_specs=[pl.BlockSpec((bn, m), lambda i: (i, 0)),
                   pl.BlockSpec((bn, 8), lambda i: (i, 0))],
        out_shape=[jax.ShapeDtypeStruct((n, m), jnp.float32),
                   jax.ShapeDtypeStruct((n, 8), jnp.float32)],
    )(x, w, wsd)


def _combine_body(acch_ref, acce_ref, s_ref, we_ref, wn_ref, wsd_ref,
                  h_ref, sd_ref):
    acc = acch_ref[0] + acch_ref[1]
    acc_e = acce_ref[0] + acce_ref[1]
    s = jnp.sum(s_ref[...], axis=1) + 1e-16
    out = (acc + jnp.dot(acc_e, we_ref[...],
                         preferred_element_type=jnp.float32)) / s[:, None]
    out = jnp.where(out > 0, out, jnp.exp(jnp.minimum(out, 0.0)) - 1.0)
    h_ref[...] = jnp.dot(out, wn_ref[...], preferred_element_type=jnp.float32)
    sd_ref[...] = jnp.dot(out, wsd_ref[...], preferred_element_type=jnp.float32)


def _combine_next(acch, acce, s, we, wn, wsd, bn=1000):
    dh = acch.shape[2]
    m = wn.shape[1]
    return pl.pallas_call(
        _combine_body,
        grid=(_N // bn,),
        in_specs=[pl.BlockSpec((2, bn, dh), lambda i: (0, i, 0)),
                  pl.BlockSpec((2, bn, 16), lambda i: (0, i, 0)),
                  pl.BlockSpec((bn, _NTILES), lambda i: (i, 0)),
                  pl.BlockSpec((16, dh), lambda i: (0, 0)),
                  pl.BlockSpec((dh, m), lambda i: (0, 0)),
                  pl.BlockSpec((dh, 8), lambda i: (0, 0))],
        out_specs=[pl.BlockSpec((bn, m), lambda i: (i, 0)),
                   pl.BlockSpec((bn, 8), lambda i: (i, 0))],
        out_shape=[jax.ShapeDtypeStruct((_N, m), jnp.float32),
                   jax.ShapeDtypeStruct((_N, 8), jnp.float32)],
    )(acch, acce, s, we, wn, wsd)


def _final_body(acch_a, acce_a, s_a, we_a, acch_c, acce_c, s_c, we_c,
                mask_ref, act_ref, lp_ref, ent_ref, val_ref, lacc, vacc):
    i = pl.program_id(0)
    ng = pl.num_programs(0)

    @pl.when(i == 0)
    def _():
        lacc[...] = jnp.zeros_like(lacc)
        vacc[...] = jnp.zeros_like(vacc)

    acc = acch_a[0] + acch_a[1]
    acc_e = acce_a[0] + acce_a[1]
    s = jnp.sum(s_a[...], axis=1) + 1e-16
    out_a = (acc + jnp.dot(acc_e, we_a[...],
                           preferred_element_type=jnp.float32)) / s[:, None]
    lacc[...] += jnp.sum(out_a, axis=0, keepdims=True)

    accc = acch_c[0] + acch_c[1]
    acc_ec = acce_c[0] + acce_c[1]
    sc = jnp.sum(s_c[...], axis=1) + 1e-16
    out_c = (accc + jnp.dot(acc_ec, we_c[...],
                            preferred_element_type=jnp.float32)) / sc[:, None]
    vacc[...] += jnp.sum(out_c, axis=0, keepdims=True)

    @pl.when(i == ng - 1)
    def _():
        logits = lacc[...] / _N
        mask = mask_ref[...] > 0
        masked = jnp.where(mask, logits, jnp.float32(-1e8))
        m = jnp.max(masked)
        lse = jnp.log(jnp.sum(jnp.exp(masked - m))) + m
        logp = masked - lse
        probs = jnp.exp(logp)
        sel = lax.broadcasted_iota(jnp.int32, logp.shape, 1) == act_ref[0]
        lp_ref[...] = jnp.full_like(lp_ref, jnp.sum(jnp.where(sel, logp, 0.0)))
        plp = jnp.where(mask, logp * probs, 0.0)
        ent_ref[...] = jnp.full_like(ent_ref, -jnp.sum(plp))
        val_ref[...] = jnp.full_like(val_ref, vacc[0, 0] / _N)


def _final_head(acch_a, acce_a, s_a, we_a, acch_c, acce_c, s_c, we_c,
                mask_f, action, bn=1000):
    z = jax.ShapeDtypeStruct((1, 1), jnp.float32)
    return pl.pallas_call(
        _final_body,
        grid=(_N // bn,),
        in_specs=[pl.BlockSpec((2, bn, 64), lambda i: (0, i, 0)),
                  pl.BlockSpec((2, bn, 16), lambda i: (0, i, 0)),
                  pl.BlockSpec((bn, _NTILES), lambda i: (i, 0)),
                  pl.BlockSpec((16, 64), lambda i: (0, 0)),
                  pl.BlockSpec((2, bn, 16), lambda i: (0, i, 0)),
                  pl.BlockSpec((2, bn, 16), lambda i: (0, i, 0)),
                  pl.BlockSpec((bn, _NTILES), lambda i: (i, 0)),
                  pl.BlockSpec((16, 16), lambda i: (0, 0)),
                  pl.BlockSpec((1, 64), lambda i: (0, 0)),
                  pl.BlockSpec(memory_space=pltpu.SMEM)],
        out_specs=[pl.BlockSpec((1, 1), lambda i: (0, 0)),
                   pl.BlockSpec((1, 1), lambda i: (0, 0)),
                   pl.BlockSpec((1, 1), lambda i: (0, 0))],
        out_shape=[z, z, z],
        scratch_shapes=[pltpu.VMEM((1, 64), jnp.float32),
                        pltpu.VMEM((1, 16), jnp.float32)],
    )(acch_a, acce_a, s_a, we_a, acch_c, acce_c, s_c, we_c, mask_f, action)


@functools.cache
def _make_sc_layer(dh):
    mesh = plsc.VectorSubcoreMesh(core_axis_name="c", subcore_axis_name="s")

    @functools.partial(
        pl.kernel,
        out_type=(jax.ShapeDtypeStruct((2, _N, dh), jnp.float32),
                  jax.ShapeDtypeStruct((2, _N, 16), jnp.float32),
                  jax.ShapeDtypeStruct((_NTILES, 1, _N), jnp.float32)),
        mesh=mesh,
        compiler_params=pltpu.CompilerParams(needs_layout_passes=False,
                                             use_tc_tiling_on_sc=False),
        scratch_types=(
            pltpu.VMEM((_N,), jnp.float32),
            pltpu.VMEM((_N,), jnp.float32),
            pltpu.VMEM((1, _N), jnp.float32),
            pltpu.VMEM((_NCHUNK, _C), jnp.int32),
            pltpu.VMEM((_NCHUNK, _C), jnp.int32),
            pltpu.VMEM((_NCHUNK, _C), jnp.float32),
            pltpu.VMEM((_C, 16), jnp.float32),
            pltpu.VMEM((_C, 16), jnp.float32),
            pltpu.VMEM((_C, dh), jnp.float32),
            pltpu.VMEM((_C, dh), jnp.float32),
            pltpu.VMEM_SHARED((_N, dh), jnp.float32),
            pltpu.VMEM_SHARED((_N, 16), jnp.float32),
            (pltpu.SemaphoreType.DMA,) * 8,
        ),
    )
    def sc_layer(h_hbm, hs_hbm, hd_hbm, src_hbm, dst_hbm, ha_hbm, ea_hbm,
                 zh_hbm, ze_hbm, z1_hbm,
                 acch_out, acce_out, s_out,
                 hs_v, hd_v, sl_v, src_v, dst_v, ha_v,
                 ea0, ea1, rows0, rows1,
                 acch_sh, acce_sh, sems):
        g0, g1, e0, e1, sh0, se0, sh1, se1 = sems
        cid = lax.axis_index("c")
        sid = lax.axis_index("s")
        wid = sid * 2 + cid

        @pl.when(sid == 0)
        def _():
            pltpu.sync_copy(zh_hbm, acch_sh)
            pltpu.sync_copy(ze_hbm, acce_sh)

        pltpu.sync_copy(hs_hbm, hs_v)
        pltpu.sync_copy(hd_hbm, hd_v)
        pltpu.sync_copy(z1_hbm, sl_v)
        pltpu.sync_copy(src_hbm.at[wid], src_v)
        pltpu.sync_copy(dst_hbm.at[wid], dst_v)
        pltpu.sync_copy(ha_hbm.at[wid], ha_v)
        plsc.subcore_barrier()

        def fire(k, rows_b, ea_b, gsem, esem):
            pltpu.async_copy(h_hbm.at[src_v.at[k]], rows_b, gsem)
            pltpu.async_copy(ea_hbm.at[pl.ds(wid * _EP + k * _C, _C)],
                             ea_b, esem)

        def wait_in(k, rows_b, ea_b, gsem, esem):
            pltpu.make_async_copy(h_hbm.at[src_v.at[k]], rows_b, gsem).wait()
            pltpu.make_async_copy(ea_hbm.at[pl.ds(wid * _EP + k * _C, _C)],
                                  ea_b, esem).wait()

        def fire_scat(k, rows_b, ea_b, shsem, sesem):
            pltpu.async_copy(rows_b, acch_sh.at[dst_v.at[k]], shsem, add=True)
            pltpu.async_copy(ea_b, acce_sh.at[dst_v.at[k]], sesem, add=True)

        def wait_scat(k, rows_b, ea_b, shsem, sesem):
            pltpu.make_async_copy(rows_b, acch_sh.at[dst_v.at[k]],
                                  shsem).wait()
            pltpu.make_async_copy(ea_b, acce_sh.at[dst_v.at[k]],
                                  sesem).wait()

        def scale_chunk(k, rows_b, ea_b):
            def scale(r, _):
                a = plsc.load_gather(
                    ha_v, [jnp.zeros((16,), jnp.int32) + k,
                           jnp.zeros((16,), jnp.int32) + r])
                for g in range(dh // 16):
                    gs = pl.ds(g * 16, 16)
                    rows_b[r, gs] = rows_b[r, gs] * a
                ea_b[r, pl.ds(0, 16)] = ea_b[r, pl.ds(0, 16)] * a
                return 0
            lax.fori_loop(0, _C, scale, 0, unroll=4)

        fire(0, rows0, ea0, g0, e0)
        fire(1, rows1, ea1, g1, e1)

        def srow(r, _):
            for j in range(_C // 16):
                sl = pl.ds(j * 16, 16)
                vsrc = src_v[r, sl]
                vdst = dst_v[r, sl]
                e = (plsc.load_gather(hs_v, [vsrc])
                     + plsc.load_gather(hd_v, [vdst])
                     + ha_v[r, sl])
                e = jnp.where(e > 0, e, 0.2 * e)
                ex = jnp.exp(e)
                ha_v[r, sl] = ex
                plsc.addupdate_scatter(
                    sl_v, [jnp.zeros((16,), jnp.int32), vdst], ex)
            return 0
        lax.fori_loop(0, _NCHUNK, srow, 0, unroll=False)

        def pair(it, _):
            k = it * 2
            wait_in(k, rows0, ea0, g0, e0)
            scale_chunk(k, rows0, ea0)
            fire_scat(k, rows0, ea0, sh0, se0)
            wait_scat(k, rows0, ea0, sh0, se0)

            @pl.when(k + 2 < _NCHUNK)
            def _():
                fire(k + 2, rows0, ea0, g0, e0)

            @pl.when(k + 1 < _NCHUNK)
            def _():
                wait_in(k + 1, rows1, ea1, g1, e1)
                scale_chunk(k + 1, rows1, ea1)
                fire_scat(k + 1, rows1, ea1, sh1, se1)
                wait_scat(k + 1, rows1, ea1, sh1, se1)

                @pl.when(k + 3 < _NCHUNK)
                def _():
                    fire(k + 3, rows1, ea1, g1, e1)
            return 0
        lax.fori_loop(0, (_NCHUNK + 1) // 2, pair, 0, unroll=False)

        pltpu.sync_copy(sl_v, s_out.at[wid])
        plsc.subcore_barrier()

        @pl.when(sid == 0)
        def _():
            pltpu.sync_copy(acch_sh, acch_out.at[cid])
            pltpu.sync_copy(acce_sh, acce_out.at[cid])

    return sc_layer


def _prep_layer(p, width=None):
    w, we = p['W'], p['We']
    asrc, adst, ae = p['a_src'], p['a_dst'], p['a_e']
    if width is not None:
        padw = width - w.shape[1]
        w = jnp.pad(w, ((0, 0), (0, padw)))
        we = jnp.pad(we, ((0, 0), (0, padw)))
        asrc = jnp.pad(asrc, (0, padw))
        adst = jnp.pad(adst, (0, padw))
    wsd = jnp.concatenate(
        [(w @ asrc)[:, None], (w @ adst)[:, None],
         jnp.zeros((w.shape[0], 6), jnp.float32)], axis=1)
    return {'W': w, 'We': we, 'wsd': wsd, 'wha': p['We'] @ p['a_e']}


def kernel(x, edge_attr, params, edge_index, invalid_action_mask, action):
    src = edge_index[0].reshape(_NTILES, _NCHUNK, _C)
    dst = edge_index[1].reshape(_NTILES, _NCHUNK, _C)

    actor = [_prep_layer(p) for p in params['actor']]
    critic = [_prep_layer(p) for p in params['critic'][:2]]
    critic.append(_prep_layer(params['critic'][2], width=16))

    w_ha = jnp.stack([l['wha'] for l in actor] + [l['wha'] for l in critic]
                     + [jnp.zeros((16,), jnp.float32)] * 2, axis=1)
    ha_all = _matmul(edge_attr, w_ha, 4000)

    zeros = {dh: jnp.zeros((_N, dh), jnp.float32) for dh in (16, 64)}
    z1 = jnp.zeros((1, _N), jnp.float32)

    def run_net(layers, base):
        h, sd = _mm2(x, layers[0]['W'], layers[0]['wsd'])
        accs = None
        for i, l in enumerate(layers):
            dh = l['W'].shape[1]
            if i > 0:
                h, sd = _combine_next(*accs, layers[i - 1]['We'],
                                      l['W'], l['wsd'])
            ha = ha_all[:, base + i].reshape(_NTILES, _NCHUNK, _C)
            acch, acce, sp = _make_sc_layer(dh)(
                h, jnp.copy(sd[:, 0]), jnp.copy(sd[:, 1]),
                src, dst, ha, edge_attr, zeros[dh], zeros[16], z1)
            accs = (acch, acce, sp.reshape(_NTILES, _N).T)
        return accs

    acch_a, acce_a, s_a = run_net(actor, 0)
    acch_c, acce_c, s_c = run_net(critic, 3)

    mask_f = invalid_action_mask.astype(jnp.float32).reshape(1, 64)
    act = jnp.asarray(action, jnp.int32).reshape(1)
    lp, ent, val = _final_head(acch_a, acce_a, s_a, actor[2]['We'],
                               acch_c, acce_c, s_c, critic[2]['We'],
                               mask_f, act)
    return (jnp.asarray(action), lp[0, 0], ent[0, 0], val[0])


# speedup vs baseline: 18.7955x; 1.0003x over previous
import functools

import jax
import jax.numpy as jnp
from jax import lax
from jax.experimental import pallas as pl
from jax.experimental.pallas import tpu as pltpu
from jax.experimental.pallas import tpu_sc as plsc

_N = 10000
_E = 320000
_C = 80
_NCHUNK = 125
_EP = _C * _NCHUNK
_NTILES = 32


def _mm_body(x_ref, w_ref, o_ref):
    o_ref[...] = jnp.dot(x_ref[...], w_ref[...],
                         preferred_element_type=jnp.float32)


def _matmul(x, w, bn):
    n, k = x.shape
    m = w.shape[1]
    return pl.pallas_call(
        _mm_body,
        grid=(n // bn,),
        in_specs=[pl.BlockSpec((bn, k), lambda i: (i, 0)),
                  pl.BlockSpec((k, m), lambda i: (0, 0))],
        out_specs=pl.BlockSpec((bn, m), lambda i: (i, 0)),
        out_shape=jax.ShapeDtypeStruct((n, m), jnp.float32),
    )(x, w)


def _mm2_body(x_ref, w_ref, wsd_ref, h_ref, sd_ref):
    x = x_ref[...]
    h_ref[...] = jnp.dot(x, w_ref[...], preferred_element_type=jnp.float32)
    sd_ref[...] = jnp.dot(x, wsd_ref[...], preferred_element_type=jnp.float32)


def _mm2(x, w, wsd, bn=1000):
    n, k = x.shape
    m = w.shape[1]
    return pl.pallas_call(
        _mm2_body,
        grid=(n // bn,),
        in_specs=[pl.BlockSpec((bn, k), lambda i: (i, 0)),
                  pl.BlockSpec((k, m), lambda i: (0, 0)),
                  pl.BlockSpec((k, 8), lambda i: (0, 0))],
        out_specs=[pl.BlockSpec((bn, m), lambda i: (i, 0)),
                   pl.BlockSpec((bn, 8), lambda i: (i, 0))],
        out_shape=[jax.ShapeDtypeStruct((n, m), jnp.float32),
                   jax.ShapeDtypeStruct((n, 8), jnp.float32)],
    )(x, w, wsd)


def _combine_body(acch_ref, acce_ref, s_ref, we_ref, wn_ref, wsd_ref,
                  h_ref, sd_ref):
    acc = acch_ref[0] + acch_ref[1]
    acc_e = acce_ref[0] + acce_ref[1]
    s = jnp.sum(s_ref[...], axis=1) + 1e-16
    out = (acc + jnp.dot(acc_e, we_ref[...],
                         preferred_element_type=jnp.float32)) / s[:, None]
    out = jnp.where(out > 0, out, jnp.exp(jnp.minimum(out, 0.0)) - 1.0)
    h_ref[...] = jnp.dot(out, wn_ref[...], preferred_element_type=jnp.float32)
    sd_ref[...] = jnp.dot(out, wsd_ref[...], preferred_element_type=jnp.float32)


def _combine_next(acch, acce, s, we, wn, wsd, bn=1000):
    dh = acch.shape[2]
    m = wn.shape[1]
    return pl.pallas_call(
        _combine_body,
        grid=(_N // bn,),
        in_specs=[pl.BlockSpec((2, bn, dh), lambda i: (0, i, 0)),
                  pl.BlockSpec((2, bn, 16), lambda i: (0, i, 0)),
                  pl.BlockSpec((bn, _NTILES), lambda i: (i, 0)),
                  pl.BlockSpec((16, dh), lambda i: (0, 0)),
                  pl.BlockSpec((dh, m), lambda i: (0, 0)),
                  pl.BlockSpec((dh, 8), lambda i: (0, 0))],
        out_specs=[pl.BlockSpec((bn, m), lambda i: (i, 0)),
                   pl.BlockSpec((bn, 8), lambda i: (i, 0))],
        out_shape=[jax.ShapeDtypeStruct((_N, m), jnp.float32),
                   jax.ShapeDtypeStruct((_N, 8), jnp.float32)],
    )(acch, acce, s, we, wn, wsd)


def _final_body(acch_a, acce_a, s_a, we_a, acch_c, acce_c, s_c, we_c,
                mask_ref, act_ref, lp_ref, ent_ref, val_ref, lacc, vacc):
    i = pl.program_id(0)
    ng = pl.num_programs(0)

    @pl.when(i == 0)
    def _():
        lacc[...] = jnp.zeros_like(lacc)
        vacc[...] = jnp.zeros_like(vacc)

    acc = acch_a[0] + acch_a[1]
    acc_e = acce_a[0] + acce_a[1]
    s = jnp.sum(s_a[...], axis=1) + 1e-16
    out_a = (acc + jnp.dot(acc_e, we_a[...],
                           preferred_element_type=jnp.float32)) / s[:, None]
    lacc[...] += jnp.sum(out_a, axis=0, keepdims=True)

    accc = acch_c[0] + acch_c[1]
    acc_ec = acce_c[0] + acce_c[1]
    sc = jnp.sum(s_c[...], axis=1) + 1e-16
    out_c = (accc + jnp.dot(acc_ec, we_c[...],
                            preferred_element_type=jnp.float32)) / sc[:, None]
    vacc[...] += jnp.sum(out_c, axis=0, keepdims=True)

    @pl.when(i == ng - 1)
    def _():
        logits = lacc[...] / _N
        mask = mask_ref[...] > 0
        masked = jnp.where(mask, logits, jnp.float32(-1e8))
        m = jnp.max(masked)
        lse = jnp.log(jnp.sum(jnp.exp(masked - m))) + m
        logp = masked - lse
        probs = jnp.exp(logp)
        sel = lax.broadcasted_iota(jnp.int32, logp.shape, 1) == act_ref[0]
        lp_ref[...] = jnp.full_like(lp_ref, jnp.sum(jnp.where(sel, logp, 0.0)))
        plp = jnp.where(mask, logp * probs, 0.0)
        ent_ref[...] = jnp.full_like(ent_ref, -jnp.sum(plp))
        val_ref[...] = jnp.full_like(val_ref, vacc[0, 0] / _N)


def _final_head(acch_a, acce_a, s_a, we_a, acch_c, acce_c, s_c, we_c,
                mask_f, action, bn=1000):
    z = jax.ShapeDtypeStruct((1, 1), jnp.float32)
    return pl.pallas_call(
        _final_body,
        grid=(_N // bn,),
        in_specs=[pl.BlockSpec((2, bn, 64), lambda i: (0, i, 0)),
                  pl.BlockSpec((2, bn, 16), lambda i: (0, i, 0)),
                  pl.BlockSpec((bn, _NTILES), lambda i: (i, 0)),
                  pl.BlockSpec((16, 64), lambda i: (0, 0)),
                  pl.BlockSpec((2, bn, 16), lambda i: (0, i, 0)),
                  pl.BlockSpec((2, bn, 16), lambda i: (0, i, 0)),
                  pl.BlockSpec((bn, _NTILES), lambda i: (i, 0)),
                  pl.BlockSpec((16, 16), lambda i: (0, 0)),
                  pl.BlockSpec((1, 64), lambda i: (0, 0)),
                  pl.BlockSpec(memory_space=pltpu.SMEM)],
        out_specs=[pl.BlockSpec((1, 1), lambda i: (0, 0)),
                   pl.BlockSpec((1, 1), lambda i: (0, 0)),
                   pl.BlockSpec((1, 1), lambda i: (0, 0))],
        out_shape=[z, z, z],
        scratch_shapes=[pltpu.VMEM((1, 64), jnp.float32),
                        pltpu.VMEM((1, 16), jnp.float32)],
    )(acch_a, acce_a, s_a, we_a, acch_c, acce_c, s_c, we_c, mask_f, action)


@functools.cache
def _make_sc_layer(dh):
    mesh = plsc.VectorSubcoreMesh(core_axis_name="c", subcore_axis_name="s")

    @functools.partial(
        pl.kernel,
        out_type=(jax.ShapeDtypeStruct((2, _N, dh), jnp.float32),
                  jax.ShapeDtypeStruct((2, _N, 16), jnp.float32),
                  jax.ShapeDtypeStruct((_NTILES, 1, _N), jnp.float32)),
        mesh=mesh,
        compiler_params=pltpu.CompilerParams(needs_layout_passes=False,
                                             use_tc_tiling_on_sc=False),
        scratch_types=(
            pltpu.VMEM((_N,), jnp.float32),
            pltpu.VMEM((_N,), jnp.float32),
            pltpu.VMEM((1, _N), jnp.float32),
            pltpu.VMEM((_NCHUNK, _C), jnp.int32),
            pltpu.VMEM((_NCHUNK, _C), jnp.int32),
            pltpu.VMEM((_NCHUNK, _C), jnp.float32),
            pltpu.VMEM((_C, 16), jnp.float32),
            pltpu.VMEM((_C, 16), jnp.float32),
            pltpu.VMEM((_C, dh), jnp.float32),
            pltpu.VMEM((_C, dh), jnp.float32),
            pltpu.VMEM_SHARED((_N, dh), jnp.float32),
            pltpu.VMEM_SHARED((_N, 16), jnp.float32),
            (pltpu.SemaphoreType.DMA,) * 8,
        ),
    )
    def sc_layer(h_hbm, hs_hbm, hd_hbm, src_hbm, dst_hbm, ha_hbm, ea_hbm,
                 zh_hbm, ze_hbm, z1_hbm,
                 acch_out, acce_out, s_out,
                 hs_v, hd_v, sl_v, src_v, dst_v, ha_v,
                 ea0, ea1, rows0, rows1,
                 acch_sh, acce_sh, sems):
        g0, g1, e0, e1, sh0, se0, sh1, se1 = sems
        cid = lax.axis_index("c")
        sid = lax.axis_index("s")
        wid = sid * 2 + cid

        @pl.when(sid == 0)
        def _():
            pltpu.sync_copy(zh_hbm, acch_sh)
            pltpu.sync_copy(ze_hbm, acce_sh)

        pltpu.sync_copy(hs_hbm, hs_v)
        pltpu.sync_copy(hd_hbm, hd_v)
        pltpu.sync_copy(z1_hbm, sl_v)
        pltpu.sync_copy(src_hbm.at[wid], src_v)
        pltpu.sync_copy(dst_hbm.at[wid], dst_v)
        pltpu.sync_copy(ha_hbm.at[wid], ha_v)
        plsc.subcore_barrier()

        def fire(k, rows_b, ea_b, gsem, esem):
            pltpu.async_copy(h_hbm.at[src_v.at[k]], rows_b, gsem)
            pltpu.async_copy(ea_hbm.at[pl.ds(wid * _EP + k * _C, _C)],
                             ea_b, esem)

        def wait_in(k, rows_b, ea_b, gsem, esem):
            pltpu.make_async_copy(h_hbm.at[src_v.at[k]], rows_b, gsem).wait()
            pltpu.make_async_copy(ea_hbm.at[pl.ds(wid * _EP + k * _C, _C)],
                                  ea_b, esem).wait()

        def fire_scat(k, rows_b, ea_b, shsem, sesem):
            pltpu.async_copy(rows_b, acch_sh.at[dst_v.at[k]], shsem, add=True)
            pltpu.async_copy(ea_b, acce_sh.at[dst_v.at[k]], sesem, add=True)

        def wait_scat(k, rows_b, ea_b, shsem, sesem):
            pltpu.make_async_copy(rows_b, acch_sh.at[dst_v.at[k]],
                                  shsem).wait()
            pltpu.make_async_copy(ea_b, acce_sh.at[dst_v.at[k]],
                                  sesem).wait()

        def scale_chunk(k, rows_b, ea_b):
            def scale(r, _):
                a = plsc.load_gather(
                    ha_v, [jnp.zeros((16,), jnp.int32) + k,
                           jnp.zeros((16,), jnp.int32) + r])
                for g in range(dh // 16):
                    gs = pl.ds(g * 16, 16)
                    rows_b[r, gs] = rows_b[r, gs] * a
                ea_b[r, pl.ds(0, 16)] = ea_b[r, pl.ds(0, 16)] * a
                return 0
            lax.fori_loop(0, _C, scale, 0, unroll=4)

        fire(0, rows0, ea0, g0, e0)
        fire(1, rows1, ea1, g1, e1)

        def srow(r, _):
            for j in range(_C // 16):
                sl = pl.ds(j * 16, 16)
                vsrc = src_v[r, sl]
                vdst = dst_v[r, sl]
                e = (plsc.load_gather(hs_v, [vsrc])
                     + plsc.load_gather(hd_v, [vdst])
                     + ha_v[r, sl])
                e = jnp.where(e > 0, e, 0.2 * e)
                ex = jnp.exp(e)
                ha_v[r, sl] = ex
                plsc.addupdate_scatter(
                    sl_v, [jnp.zeros((16,), jnp.int32), vdst], ex)
            return 0
        lax.fori_loop(0, _NCHUNK, srow, 0, unroll=False)

        def pair(it, _):
            k = it * 2
            wait_in(k, rows0, ea0, g0, e0)
            scale_chunk(k, rows0, ea0)
            fire_scat(k, rows0, ea0, sh0, se0)
            wait_scat(k, rows0, ea0, sh0, se0)

            @pl.when(k + 2 < _NCHUNK)
            def _():
                fire(k + 2, rows0, ea0, g0, e0)

            @pl.when(k + 1 < _NCHUNK)
            def _():
                wait_in(k + 1, rows1, ea1, g1, e1)
                scale_chunk(k + 1, rows1, ea1)
                fire_scat(k + 1, rows1, ea1, sh1, se1)
                wait_scat(k + 1, rows1, ea1, sh1, se1)

                @pl.when(k + 3 < _NCHUNK)
                def _():
                    fire(k + 3, rows1, ea1, g1, e1)
            return 0
        lax.fori_loop(0, (_NCHUNK + 1) // 2, pair, 0, unroll=False)

        pltpu.sync_copy(sl_v, s_out.at[wid])
        plsc.subcore_barrier()

        @pl.when(sid == 0)
        def _():
            pltpu.sync_copy(acch_sh, acch_out.at[cid])
            pltpu.sync_copy(acce_sh, acce_out.at[cid])

    return sc_layer


def _prep_layer(p, width=None):
    w, we = p['W'], p['We']
    asrc, adst, ae = p['a_src'], p['a_dst'], p['a_e']
    if width is not None:
        padw = width - w.shape[1]
        w = jnp.pad(w, ((0, 0), (0, padw)))
        we = jnp.pad(we, ((0, 0), (0, padw)))
        asrc = jnp.pad(asrc, (0, padw))
        adst = jnp.pad(adst, (0, padw))
    wsd = jnp.concatenate(
        [(w @ asrc)[:, None], (w @ adst)[:, None],
         jnp.zeros((w.shape[0], 6), jnp.float32)], axis=1)
    return {'W': w, 'We': we, 'wsd': wsd, 'wha': p['We'] @ p['a_e']}


def kernel(x, edge_attr, params, edge_index, invalid_action_mask, action):
    src = edge_index[0].reshape(_NTILES, _NCHUNK, _C)
    dst = edge_index[1].reshape(_NTILES, _NCHUNK, _C)

    actor = [_prep_layer(p) for p in params['actor']]
    critic = [_prep_layer(p) for p in params['critic'][:2]]
    critic.append(_prep_layer(params['critic'][2], width=16))

    w_ha = jnp.stack([l['wha'] for l in actor] + [l['wha'] for l in critic]
                     + [jnp.zeros((16,), jnp.float32)] * 2, axis=1)
    ha_all = _matmul(edge_attr, w_ha, 4000)

    zeros = {dh: jnp.zeros((_N, dh), jnp.float32) for dh in (16, 64)}
    z1 = jnp.zeros((1, _N), jnp.float32)

    def sc_step(l, h, sd, base_i):
        dh = l['W'].shape[1]
        ha = ha_all[:, base_i].reshape(_NTILES, _NCHUNK, _C)
        acch, acce, sp = _make_sc_layer(dh)(
            h, jnp.copy(sd[:, 0]), jnp.copy(sd[:, 1]),
            src, dst, ha, edge_attr, zeros[dh], zeros[16], z1)
        return (acch, acce, sp.reshape(_NTILES, _N).T)

    hsd = {'a': _mm2(x, actor[0]['W'], actor[0]['wsd']),
           'c': _mm2(x, critic[0]['W'], critic[0]['wsd'])}
    accs = {}
    for i in range(3):
        if i > 0:
            hsd['a'] = _combine_next(*accs['a'], actor[i - 1]['We'],
                                     actor[i]['W'], actor[i]['wsd'])
            hsd['c'] = _combine_next(*accs['c'], critic[i - 1]['We'],
                                     critic[i]['W'], critic[i]['wsd'])
        accs['a'] = sc_step(actor[i], *hsd['a'], i)
        accs['c'] = sc_step(critic[i], *hsd['c'], 3 + i)

    acch_a, acce_a, s_a = accs['a']
    acch_c, acce_c, s_c = accs['c']

    mask_f = invalid_action_mask.astype(jnp.float32).reshape(1, 64)
    act = jnp.asarray(action, jnp.int32).reshape(1)
    lp, ent, val = _final_head(acch_a, acce_a, s_a, actor[2]['We'],
                               acch_c, acce_c, s_c, critic[2]['We'],
                               mask_f, act)
    return (jnp.asarray(action), lp[0, 0], ent[0, 0], val[0])
